# baseline (speedup 1.0000x reference)
import functools

import jax
import jax.numpy as jnp
from jax import lax
from jax.experimental import pallas as pl
from jax.experimental.pallas import tpu as pltpu
from jax.experimental.pallas import tpu_sc as plsc

N = 10000
NP = 10240
D = 256
DH = 128
NSUB = 16
B = 128
CPT = 80
PER_TILE = CPT * B
E_PAD = NSUB * PER_TILE
ROWS_PER_TILE = NP // NSUB
WB = 5
WROWS = ROWS_PER_TILE // WB
RB = 512
NRB = NP // RB
SEGS = 2
CPS = CPT // SEGS
SEGE = CPS * B


def _mm1_kernel(x_ref, w_ref, o_ref):
    o_ref[...] = jnp.dot(x_ref[...], w_ref[...],
                         preferred_element_type=jnp.float32)


def _mm2_kernel(a_ref, w_ref, o_ref):
    a = jnp.maximum(a_ref[...], 0.0)
    o_ref[...] = jnp.dot(a, w_ref[...], preferred_element_type=jnp.float32)


def _mm1(x, w):
    return pl.pallas_call(
        _mm1_kernel,
        grid=(NRB, 2),
        in_specs=[
            pl.BlockSpec((RB, D), lambda i, j: (i, 0)),
            pl.BlockSpec((D, DH), lambda i, j: (0, j)),
        ],
        out_specs=pl.BlockSpec((RB, DH), lambda i, j: (j * NRB + i, 0)),
        out_shape=jax.ShapeDtypeStruct((2 * NP, DH), jnp.float32),
    )(x, w)


def _mm2(a2, w):
    return pl.pallas_call(
        _mm2_kernel,
        grid=(NRB, 2),
        in_specs=[
            pl.BlockSpec((RB, D), lambda i, j: (i, 0)),
            pl.BlockSpec((D, DH), lambda i, j: (0, j)),
        ],
        out_specs=pl.BlockSpec((RB, DH), lambda i, j: (j * NRB + i, 0)),
        out_shape=jax.ShapeDtypeStruct((2 * NP, DH), jnp.float32),
    )(a2, w)


_mesh = plsc.VectorSubcoreMesh(core_axis_name="c", subcore_axis_name="s")


@functools.partial(
    pl.kernel,
    mesh=_mesh,
    out_type=jax.ShapeDtypeStruct((NP, D), jnp.float32),
    scratch_types=[
        pltpu.VMEM((SEGE,), jnp.int32),
        pltpu.VMEM((CPS, B), jnp.int32),
        pltpu.VMEM((SEGE,), jnp.float32),
        pltpu.VMEM((2, B, DH), jnp.float32),
        pltpu.VMEM_SHARED((NP, DH), jnp.float32),
        pltpu.SemaphoreType.DMA,
        pltpu.SemaphoreType.DMA,
        pltpu.SemaphoreType.DMA,
        pltpu.SemaphoreType.DMA,
    ],
)
def _agg(t2, src2, dst3, attr, out, src2_v, dst_v, attr_v, rows_v, acc,
         sg0, sg1, ss0, ss1):
    c = lax.axis_index("c")
    s = lax.axis_index("s")
    sem_g = (sg0, sg1)
    sem_s = (ss0, ss1)
    ebase = s * PER_TILE

    def _gather_desc(k, b):
        return pltpu.make_async_copy(
            t2.at[src2_v.at[pl.ds(k * B, B)]], rows_v.at[b], sem_g[b])

    def _scatter_desc(k, b):
        return pltpu.make_async_copy(
            rows_v.at[b], acc.at[dst_v.at[k]], sem_s[b])

    def _zrow(i, _):
        for j in range(DH // 16):
            rows_v[0, i, pl.ds(j * 16, 16)] = jnp.zeros((16,), jnp.float32)
        return 0

    lax.fori_loop(0, B, _zrow, 0)
    for w in range(WB):
        r = pl.multiple_of(s * ROWS_PER_TILE + w * WROWS, 8)
        pltpu.sync_copy(rows_v.at[0], acc.at[pl.ds(r, WROWS)])

    for seg in range(SEGS):
        so = pl.multiple_of(ebase + seg * SEGE, 8)
        if seg > 0:
            _scatter_desc(CPS - 2, 0).wait()
            _scatter_desc(CPS - 1, 1).wait()
        pltpu.sync_copy(
            src2.at[pl.ds(pl.multiple_of(c * E_PAD + so, 8), SEGE)], src2_v)
        pltpu.sync_copy(dst3.at[s].at[pl.ds(seg * CPS, CPS)], dst_v)
        pltpu.sync_copy(attr.at[pl.ds(so, SEGE)], attr_v)
        _gather_desc(0, 0).start()
        if seg == 0:
            plsc.subcore_barrier()

        def _pair(p, _):
            for b in range(2):
                k = p * 2 + b
                nb = 1 - b

                @pl.when(k + 1 < CPS)
                def _():
                    @pl.when(k >= 1)
                    def _():
                        _scatter_desc(k - 1, nb).wait()
                    _gather_desc(k + 1, nb).start()

                _gather_desc(k, b).wait()

                def _grp(g, _):
                    av = attr_v[pl.ds(k * B + g * 16, 16)]
                    for u in range(16):
                        sp = jnp.take_along_axis(
                            av, jnp.full((16,), u, jnp.int32), axis=0)
                        ii = g * 16 + u
                        for j in range(DH // 16):
                            sl = pl.ds(j * 16, 16)
                            rows_v[b, ii, sl] = rows_v[b, ii, sl] * sp
                    return 0

                lax.fori_loop(0, B // 16, _grp, 0)
                _scatter_desc(k, b).start(add=True)
            return 0

        lax.fori_loop(0, CPS // 2, _pair, 0)

    _scatter_desc(CPS - 2, 0).wait()
    _scatter_desc(CPS - 1, 1).wait()
    plsc.subcore_barrier()

    cc = pl.multiple_of(c * DH, 8)
    for w in range(WB):
        r = pl.multiple_of(s * ROWS_PER_TILE + w * WROWS, 8)
        pltpu.sync_copy(acc.at[pl.ds(r, WROWS)], rows_v.at[0])
        pltpu.sync_copy(rows_v.at[0],
                        out.at[pl.ds(r, WROWS), pl.ds(cc, DH)])


def kernel(x, edge_index, edge_attr, batch, W1, W2):
    src = edge_index[0].astype(jnp.int32)
    dst = edge_index[1].astype(jnp.int32)
    attr = edge_attr.astype(jnp.float32)
    pad = E_PAD - src.shape[0]
    zi = jnp.zeros((pad,), jnp.int32)
    src_p = jnp.concatenate([src, zi])
    dst_p = jnp.concatenate([dst, zi])
    attr_p = jnp.concatenate([attr, jnp.zeros((pad,), jnp.float32)])
    dst3 = dst_p.reshape(NSUB, CPT, B)
    src2 = jnp.concatenate([src_p, src_p + NP])

    t2 = _mm1(x, W1)
    y2 = _agg(t2, src2, dst3, attr_p)
    u2 = _mm2(y2, W2)
    o2 = _agg(u2, src2, dst3, attr_p)
    return o2[:N]


# speedup vs baseline: 1.0007x; 1.0007x over previous
import functools

import jax
import jax.numpy as jnp
from jax import lax
from jax.experimental import pallas as pl
from jax.experimental.pallas import tpu as pltpu
from jax.experimental.pallas import tpu_sc as plsc

N = 10000
NP = 10240
D = 256
DH = 128
NSUB = 16
B = 128
CPT = 80
PER_TILE = CPT * B
E_PAD = NSUB * PER_TILE
ROWS_PER_TILE = NP // NSUB
WB = 5
WROWS = ROWS_PER_TILE // WB
RB = 512
NRB = NP // RB
SEGS = 2
CPS = CPT // SEGS
SEGE = CPS * B


def _mm1_kernel(x_ref, w_ref, o_ref):
    o_ref[...] = jnp.dot(x_ref[...], w_ref[...],
                         preferred_element_type=jnp.float32)


def _mm2_kernel(a_ref, w_ref, o_ref):
    a = jnp.maximum(a_ref[...], 0.0)
    o_ref[...] = jnp.dot(a, w_ref[...], preferred_element_type=jnp.float32)


def _mm1(x, w):
    return pl.pallas_call(
        _mm1_kernel,
        grid=(NRB, 2),
        in_specs=[
            pl.BlockSpec((RB, D), lambda i, j: (i, 0)),
            pl.BlockSpec((D, DH), lambda i, j: (0, j)),
        ],
        out_specs=pl.BlockSpec((RB, DH), lambda i, j: (j * NRB + i, 0)),
        out_shape=jax.ShapeDtypeStruct((2 * NP, DH), jnp.float32),
    )(x, w)


def _mm2(a2, w):
    return pl.pallas_call(
        _mm2_kernel,
        grid=(NRB, 2),
        in_specs=[
            pl.BlockSpec((RB, D), lambda i, j: (i, 0)),
            pl.BlockSpec((D, DH), lambda i, j: (0, j)),
        ],
        out_specs=pl.BlockSpec((RB, DH), lambda i, j: (j * NRB + i, 0)),
        out_shape=jax.ShapeDtypeStruct((2 * NP, DH), jnp.float32),
    )(a2, w)


_mesh = plsc.VectorSubcoreMesh(core_axis_name="c", subcore_axis_name="s")


@functools.partial(
    pl.kernel,
    mesh=_mesh,
    out_type=jax.ShapeDtypeStruct((NP, D), jnp.float32),
    scratch_types=[
        pltpu.VMEM((SEGE,), jnp.int32),
        pltpu.VMEM((CPS, B), jnp.int32),
        pltpu.VMEM((SEGE,), jnp.float32),
        pltpu.VMEM((2, B, DH), jnp.float32),
        pltpu.VMEM_SHARED((NP, DH), jnp.float32),
        pltpu.SemaphoreType.DMA,
        pltpu.SemaphoreType.DMA,
        pltpu.SemaphoreType.DMA,
        pltpu.SemaphoreType.DMA,
        pltpu.SemaphoreType.DMA,
        pltpu.SemaphoreType.DMA,
    ],
)
def _agg(t2, src2, dst3, attr, out, src2_v, dst_v, attr_v, rows_v, acc,
         sg0, sg1, sh0, sh1, ss0, ss1):
    c = lax.axis_index("c")
    s = lax.axis_index("s")
    sem_g = (sg0, sg1)
    sem_h = (sh0, sh1)
    sem_s = (ss0, ss1)
    ebase = s * PER_TILE
    HB = B // 2

    def _gather_lo(k, b):
        return pltpu.make_async_copy(
            t2.at[src2_v.at[pl.ds(k * B, HB)]],
            rows_v.at[b].at[pl.ds(0, HB)], sem_g[b])

    def _gather_hi(k, b):
        return pltpu.make_async_copy(
            t2.at[src2_v.at[pl.ds(k * B + HB, HB)]],
            rows_v.at[b].at[pl.ds(HB, HB)], sem_h[b])

    def _gather_start(k, b):
        _gather_lo(k, b).start()
        _gather_hi(k, b).start()

    def _gather_wait(k, b):
        _gather_lo(k, b).wait()
        _gather_hi(k, b).wait()

    def _scatter_desc(k, b):
        return pltpu.make_async_copy(
            rows_v.at[b], acc.at[dst_v.at[k]], sem_s[b])

    def _zrow(i, _):
        for j in range(DH // 16):
            rows_v[0, i, pl.ds(j * 16, 16)] = jnp.zeros((16,), jnp.float32)
        return 0

    lax.fori_loop(0, B, _zrow, 0)
    for w in range(WB):
        r = pl.multiple_of(s * ROWS_PER_TILE + w * WROWS, 8)
        pltpu.sync_copy(rows_v.at[0], acc.at[pl.ds(r, WROWS)])

    for seg in range(SEGS):
        so = pl.multiple_of(ebase + seg * SEGE, 8)
        if seg > 0:
            _scatter_desc(CPS - 2, 0).wait()
            _scatter_desc(CPS - 1, 1).wait()
        pltpu.sync_copy(
            src2.at[pl.ds(pl.multiple_of(c * E_PAD + so, 8), SEGE)], src2_v)
        pltpu.sync_copy(dst3.at[s].at[pl.ds(seg * CPS, CPS)], dst_v)
        pltpu.sync_copy(attr.at[pl.ds(so, SEGE)], attr_v)
        _gather_start(0, 0)
        if seg == 0:
            plsc.subcore_barrier()

        def _pair(p, _):
            for b in range(2):
                k = p * 2 + b
                nb = 1 - b

                @pl.when(k + 1 < CPS)
                def _():
                    @pl.when(k >= 1)
                    def _():
                        _scatter_desc(k - 1, nb).wait()
                    _gather_start(k + 1, nb)

                _gather_wait(k, b)

                def _grp(g, _):
                    av = attr_v[pl.ds(k * B + g * 16, 16)]
                    for u in range(16):
                        sp = jnp.take_along_axis(
                            av, jnp.full((16,), u, jnp.int32), axis=0)
                        ii = g * 16 + u
                        for j in range(DH // 16):
                            sl = pl.ds(j * 16, 16)
                            rows_v[b, ii, sl] = rows_v[b, ii, sl] * sp
                    return 0

                lax.fori_loop(0, B // 16, _grp, 0)
                _scatter_desc(k, b).start(add=True)
            return 0

        lax.fori_loop(0, CPS // 2, _pair, 0)

    _scatter_desc(CPS - 2, 0).wait()
    _scatter_desc(CPS - 1, 1).wait()
    plsc.subcore_barrier()

    cc = pl.multiple_of(c * DH, 8)
    for w in range(WB):
        r = pl.multiple_of(s * ROWS_PER_TILE + w * WROWS, 8)
        pltpu.sync_copy(acc.at[pl.ds(r, WROWS)], rows_v.at[0])
        pltpu.sync_copy(rows_v.at[0],
                        out.at[pl.ds(r, WROWS), pl.ds(cc, DH)])


def kernel(x, edge_index, edge_attr, batch, W1, W2):
    src = edge_index[0].astype(jnp.int32)
    dst = edge_index[1].astype(jnp.int32)
    attr = edge_attr.astype(jnp.float32)
    pad = E_PAD - src.shape[0]
    zi = jnp.zeros((pad,), jnp.int32)
    src_p = jnp.concatenate([src, zi])
    dst_p = jnp.concatenate([dst, zi])
    attr_p = jnp.concatenate([attr, jnp.zeros((pad,), jnp.float32)])
    dst3 = dst_p.reshape(NSUB, CPT, B)
    src2 = jnp.concatenate([src_p, src_p + NP])

    t2 = _mm1(x, W1)
    y2 = _agg(t2, src2, dst3, attr_p)
    u2 = _mm2(y2, W2)
    o2 = _agg(u2, src2, dst3, attr_p)
    return o2[:N]
